# baseline (speedup 1.0000x reference)
import functools

import jax
import jax.numpy as jnp
from jax import lax
from jax.experimental import pallas as pl
from jax.experimental.pallas import tpu as pltpu
from jax.experimental.pallas import tpu_sc as plsc

KP = 1_048_576
ROWS_P = 2048
COLS_P = 1024
NEG = -9999.0
BK = 32768
CH = 8192
L = 16


def _mlp_body(inp, i0, i1, w1, b1, w2, b2, w3, b3, w4, b4, xo, co):
    a = inp[...]
    h = jnp.maximum(jnp.dot(w1[...], a, preferred_element_type=jnp.float32) + b1[...], 0.0)
    h = jnp.maximum(jnp.dot(w2[...], h, preferred_element_type=jnp.float32) + b2[...], 0.0)
    h = jnp.maximum(jnp.dot(w3[...], h, preferred_element_type=jnp.float32) + b3[...], 0.0)
    xo[...] = (jnp.dot(w4[...], h, preferred_element_type=jnp.float32) + b4[...]).reshape(BK)
    co[...] = (i0[...] * COLS_P + i1[...]).reshape(BK)


def _mlp_call(inp_p, i0, i1, W1, b1, W2, b2, W3, b3, W4, b4):
    grid = KP // BK
    full = lambda i: (0, 0)
    lane = lambda i: (0, i)
    return pl.pallas_call(
        _mlp_body,
        grid=(grid,),
        in_specs=[
            pl.BlockSpec((9, BK), lane),
            pl.BlockSpec((1, BK), lane),
            pl.BlockSpec((1, BK), lane),
            pl.BlockSpec((18, 9), full),
            pl.BlockSpec((18, 1), full),
            pl.BlockSpec((36, 18), full),
            pl.BlockSpec((36, 1), full),
            pl.BlockSpec((36, 36), full),
            pl.BlockSpec((36, 1), full),
            pl.BlockSpec((1, 36), full),
            pl.BlockSpec((1, 1), full),
        ],
        out_specs=[pl.BlockSpec((BK,), lambda i: (i,)),
                   pl.BlockSpec((BK,), lambda i: (i,))],
        out_shape=[
            jax.ShapeDtypeStruct((KP,), jnp.float32),
            jax.ShapeDtypeStruct((KP,), jnp.int32),
        ],
    )(inp_p, i0, i1, W1, b1.reshape(18, 1), W2, b2.reshape(36, 1),
      W3, b3.reshape(36, 1), W4, b4.reshape(1, 1))


@functools.cache
def _make_sc_scatter(NC, NS):
    NW = NC * NS
    CELLS_PER = (ROWS_P * COLS_P) // NW
    NROWS = ROWS_P // NW
    NCH = KP // CH
    NB = 2
    mesh = plsc.VectorSubcoreMesh(core_axis_name="c", subcore_axis_name="s")

    @functools.partial(
        pl.kernel,
        out_type=(jax.ShapeDtypeStruct((ROWS_P,), jnp.float32),
                  jax.ShapeDtypeStruct((NW, COLS_P), jnp.float32)),
        mesh=mesh,
        scratch_types=[
            pltpu.VMEM((CELLS_PER,), jnp.float32),
            pltpu.VMEM((CH,), jnp.int32),
            pltpu.VMEM((CH,), jnp.int32),
            pltpu.VMEM((CH,), jnp.float32),
            pltpu.VMEM((CH,), jnp.float32),
            pltpu.VMEM((COLS_P,), jnp.float32),
            pltpu.VMEM((NROWS,), jnp.float32),
            pltpu.SemaphoreType.DMA,
            pltpu.SemaphoreType.DMA,
            pltpu.SemaphoreType.DMA,
            pltpu.SemaphoreType.DMA,
        ],
        compiler_params=pltpu.CompilerParams(needs_layout_passes=False),
    )
    def sc_a(cell_hbm, x_hbm, out1_hbm, colpart_hbm,
             vtab, cbuf0, cbuf1, xbuf0, xbuf1, colp, rowm,
             csem0, csem1, xsem0, xsem1):
        w = lax.axis_index("c") * NS + lax.axis_index("s")
        base = w * CELLS_PER
        iota = lax.iota(jnp.int32, L)
        negs = jnp.full((L,), NEG, jnp.float32)

        def init_body(i, _):
            vtab[pl.ds(i * L, L)] = negs
            return 0
        lax.fori_loop(0, CELLS_PER // L, init_body, 0)

        cbufs = (cbuf0, cbuf1)
        xbufs = (xbuf0, xbuf1)
        csems = (csem0, csem1)
        xsems = (xsem0, xsem1)
        for b in range(NB):
            pltpu.async_copy(cell_hbm.at[pl.ds(b * CH, CH)], cbufs[b], csems[b])
            pltpu.async_copy(x_hbm.at[pl.ds(b * CH, CH)], xbufs[b], xsems[b])

        UN = 8

        def process(cbuf, xbuf):
            def body(i, _):
                for u in range(UN):
                    j = i * UN + u
                    cv = cbuf[pl.ds(j * L, L)]
                    xv = xbuf[pl.ds(j * L, L)]
                    lv = cv - base
                    valid = lv.astype(jnp.uint32) < CELLS_PER
                    plsc.store_scatter(vtab, [lv], xv, mask=valid)
                return 0
            lax.fori_loop(0, CH // (L * UN), body, 0)

        def outer(g, _):
            for b in range(NB):
                ch = g * NB + b
                pltpu.make_async_copy(
                    cell_hbm.at[pl.ds(ch * CH, CH)], cbufs[b], csems[b]).wait()
                pltpu.make_async_copy(
                    x_hbm.at[pl.ds(ch * CH, CH)], xbufs[b], xsems[b]).wait()
                process(cbufs[b], xbufs[b])

                @pl.when(ch + NB < NCH)
                def _next():
                    pltpu.async_copy(
                        cell_hbm.at[pl.ds((ch + NB) * CH, CH)], cbufs[b], csems[b])
                    pltpu.async_copy(
                        x_hbm.at[pl.ds((ch + NB) * CH, CH)], xbufs[b], xsems[b])
            return 0
        lax.fori_loop(0, NCH // NB, outer, 0)

        def cinit(i, _):
            colp[pl.ds(i * L, L)] = negs
            return 0
        lax.fori_loop(0, COLS_P // L, cinit, 0)

        RUN = 4

        def row_body(r, _):
            def mbody(i, acc):
                accs = []
                for u in range(RUN):
                    j = i * RUN + u
                    vv = vtab[pl.ds(r * COLS_P + j * L, L)]
                    colp[pl.ds(j * L, L)] = jnp.maximum(colp[pl.ds(j * L, L)], vv)
                    accs.append(vv)
                m = jnp.maximum(jnp.maximum(accs[0], accs[1]),
                                jnp.maximum(accs[2], accs[3]))
                return jnp.maximum(acc, m)
            rm = lax.fori_loop(0, COLS_P // (L * RUN), mbody, negs)
            plsc.store_scatter(rowm, [jnp.full((L,), r, jnp.int32)],
                               jnp.full((L,), jnp.max(rm), jnp.float32),
                               mask=iota == 0)
            return 0
        lax.fori_loop(0, NROWS, row_body, 0)

        pltpu.sync_copy(rowm, out1_hbm.at[pl.ds(w * NROWS, NROWS)])
        pltpu.sync_copy(colp, colpart_hbm.at[w])

    return sc_a


def _colmax_body(cp, o):
    o[...] = jnp.max(cp[...], axis=0, keepdims=True)


def kernel(input, T_out, T_indices, W1, b1, W2, b2, W3, b3, W4, b4):
    K = input.shape[-1]
    pad = KP - K
    inp_p = jnp.pad(input.reshape(9, K), ((0, 0), (0, pad)))
    idx = T_indices.astype(jnp.int32)
    i0 = jnp.pad(idx[0], (0, pad), constant_values=ROWS_P - 1).reshape(1, KP)
    i1 = jnp.pad(idx[1], (0, pad), constant_values=COLS_P - 1).reshape(1, KP)

    x, cells = _mlp_call(inp_p, i0, i1, W1, b1, W2, b2, W3, b3, W4, b4)

    info = plsc.get_sparse_core_info()
    sc_a = _make_sc_scatter(info.num_cores, info.num_subcores)
    out1p, colpart = sc_a(cells, x)

    out2p = pl.pallas_call(
        _colmax_body,
        out_shape=jax.ShapeDtypeStruct((1, COLS_P), jnp.float32),
    )(colpart)

    n1 = T_out.shape[1]
    n2 = T_out.shape[2]
    return out1p[:n1], out2p[0, :n2]


# speedup vs baseline: 1.0659x; 1.0659x over previous
import functools

import jax
import jax.numpy as jnp
from jax import lax
from jax.experimental import pallas as pl
from jax.experimental.pallas import tpu as pltpu
from jax.experimental.pallas import tpu_sc as plsc

KP = 1_015_808
ROWS_P = 2048
COLS_P = 1024
NEG = -9999.0
BK = 32768
CH = 8192
L = 16


PADCELL = ROWS_P * COLS_P - 1


def _mlp_body(K, inp, i0, i1, w1, b1, w2, b2, w3, b3, w4, b4, xo, co):
    a = inp[...]
    h = jnp.maximum(jnp.dot(w1[...], a, preferred_element_type=jnp.float32) + b1[...], 0.0)
    h = jnp.maximum(jnp.dot(w2[...], h, preferred_element_type=jnp.float32) + b2[...], 0.0)
    h = jnp.maximum(jnp.dot(w3[...], h, preferred_element_type=jnp.float32) + b3[...], 0.0)
    xo[...] = (jnp.dot(w4[...], h, preferred_element_type=jnp.float32) + b4[...]).reshape(BK)
    pos = pl.program_id(0) * BK + lax.broadcasted_iota(jnp.int32, (1, BK), 1)
    cell = i0[...] * COLS_P + i1[...]
    co[...] = jnp.where(pos < K, cell, PADCELL).reshape(BK)


def _mlp_call(inp_p, i0, i1, W1, b1, W2, b2, W3, b3, W4, b4):
    grid = KP // BK
    full = lambda i: (0, 0)
    lane = lambda i: (0, i)
    K = inp_p.shape[-1]
    return pl.pallas_call(
        functools.partial(_mlp_body, K),
        grid=(grid,),
        in_specs=[
            pl.BlockSpec((9, BK), lane),
            pl.BlockSpec((1, BK), lane),
            pl.BlockSpec((1, BK), lane),
            pl.BlockSpec((18, 9), full),
            pl.BlockSpec((18, 1), full),
            pl.BlockSpec((36, 18), full),
            pl.BlockSpec((36, 1), full),
            pl.BlockSpec((36, 36), full),
            pl.BlockSpec((36, 1), full),
            pl.BlockSpec((1, 36), full),
            pl.BlockSpec((1, 1), full),
        ],
        out_specs=[pl.BlockSpec((BK,), lambda i: (i,)),
                   pl.BlockSpec((BK,), lambda i: (i,))],
        out_shape=[
            jax.ShapeDtypeStruct((KP,), jnp.float32),
            jax.ShapeDtypeStruct((KP,), jnp.int32),
        ],
    )(inp_p, i0, i1, W1, b1.reshape(18, 1), W2, b2.reshape(36, 1),
      W3, b3.reshape(36, 1), W4, b4.reshape(1, 1))


@functools.cache
def _make_sc_scatter(NC, NS):
    NW = NC * NS
    CELLS_PER = (ROWS_P * COLS_P) // NW
    NROWS = ROWS_P // NW
    NCH = KP // CH
    NB = 2
    mesh = plsc.VectorSubcoreMesh(core_axis_name="c", subcore_axis_name="s")

    @functools.partial(
        pl.kernel,
        out_type=(jax.ShapeDtypeStruct((ROWS_P,), jnp.float32),
                  jax.ShapeDtypeStruct((NW, COLS_P), jnp.float32)),
        mesh=mesh,
        scratch_types=[
            pltpu.VMEM((CELLS_PER,), jnp.float32),
            pltpu.VMEM((CH,), jnp.int32),
            pltpu.VMEM((CH,), jnp.int32),
            pltpu.VMEM((CH,), jnp.float32),
            pltpu.VMEM((CH,), jnp.float32),
            pltpu.VMEM((COLS_P,), jnp.float32),
            pltpu.VMEM((NROWS,), jnp.float32),
            pltpu.SemaphoreType.DMA,
            pltpu.SemaphoreType.DMA,
            pltpu.SemaphoreType.DMA,
            pltpu.SemaphoreType.DMA,
        ],
        compiler_params=pltpu.CompilerParams(needs_layout_passes=False),
    )
    def sc_a(cell_hbm, x_hbm, out1_hbm, colpart_hbm,
             vtab, cbuf0, cbuf1, xbuf0, xbuf1, colp, rowm,
             csem0, csem1, xsem0, xsem1):
        w = lax.axis_index("c") * NS + lax.axis_index("s")
        base = w * CELLS_PER
        iota = lax.iota(jnp.int32, L)
        negs = jnp.full((L,), NEG, jnp.float32)

        def init_body(i, _):
            vtab[pl.ds(i * L, L)] = negs
            return 0
        lax.fori_loop(0, CELLS_PER // L, init_body, 0)

        cbufs = (cbuf0, cbuf1)
        xbufs = (xbuf0, xbuf1)
        csems = (csem0, csem1)
        xsems = (xsem0, xsem1)
        for b in range(NB):
            pltpu.async_copy(cell_hbm.at[pl.ds(b * CH, CH)], cbufs[b], csems[b])
            pltpu.async_copy(x_hbm.at[pl.ds(b * CH, CH)], xbufs[b], xsems[b])

        UN = 8

        def process(cbuf, xbuf):
            def body(i, _):
                for u in range(UN):
                    j = i * UN + u
                    cv = cbuf[pl.ds(j * L, L)]
                    xv = xbuf[pl.ds(j * L, L)]
                    lv = cv - base
                    valid = lv.astype(jnp.uint32) < CELLS_PER
                    plsc.store_scatter(vtab, [lv], xv, mask=valid)
                return 0
            lax.fori_loop(0, CH // (L * UN), body, 0)

        def outer(g, _):
            for b in range(NB):
                ch = g * NB + b
                pltpu.make_async_copy(
                    cell_hbm.at[pl.ds(ch * CH, CH)], cbufs[b], csems[b]).wait()
                pltpu.make_async_copy(
                    x_hbm.at[pl.ds(ch * CH, CH)], xbufs[b], xsems[b]).wait()
                process(cbufs[b], xbufs[b])

                @pl.when(ch + NB < NCH)
                def _next():
                    pltpu.async_copy(
                        cell_hbm.at[pl.ds((ch + NB) * CH, CH)], cbufs[b], csems[b])
                    pltpu.async_copy(
                        x_hbm.at[pl.ds((ch + NB) * CH, CH)], xbufs[b], xsems[b])
            return 0
        lax.fori_loop(0, NCH // NB, outer, 0)

        def cinit(i, _):
            colp[pl.ds(i * L, L)] = negs
            return 0
        lax.fori_loop(0, COLS_P // L, cinit, 0)

        RUN = 4

        def row_body(r, _):
            def mbody(i, acc):
                accs = []
                for u in range(RUN):
                    j = i * RUN + u
                    vv = vtab[pl.ds(r * COLS_P + j * L, L)]
                    colp[pl.ds(j * L, L)] = jnp.maximum(colp[pl.ds(j * L, L)], vv)
                    accs.append(vv)
                m = jnp.maximum(jnp.maximum(accs[0], accs[1]),
                                jnp.maximum(accs[2], accs[3]))
                return jnp.maximum(acc, m)
            rm = lax.fori_loop(0, COLS_P // (L * RUN), mbody, negs)
            plsc.store_scatter(rowm, [jnp.full((L,), r, jnp.int32)],
                               jnp.full((L,), jnp.max(rm), jnp.float32),
                               mask=iota == 0)
            return 0
        lax.fori_loop(0, NROWS, row_body, 0)

        pltpu.sync_copy(rowm, out1_hbm.at[pl.ds(w * NROWS, NROWS)])
        pltpu.sync_copy(colp, colpart_hbm.at[w])

    return sc_a


def _colmax_body(cp, o):
    o[...] = jnp.max(cp[...], axis=0, keepdims=True)


def kernel(input, T_out, T_indices, W1, b1, W2, b2, W3, b3, W4, b4):
    K = input.shape[-1]
    inp_p = input.reshape(9, K)
    idx = T_indices.astype(jnp.int32)
    i0 = idx[0].reshape(1, K)
    i1 = idx[1].reshape(1, K)

    x, cells = _mlp_call(inp_p, i0, i1, W1, b1, W2, b2, W3, b3, W4, b4)

    info = plsc.get_sparse_core_info()
    sc_a = _make_sc_scatter(info.num_cores, info.num_subcores)
    out1p, colpart = sc_a(cells, x)

    out2p = pl.pallas_call(
        _colmax_body,
        out_shape=jax.ShapeDtypeStruct((1, COLS_P), jnp.float32),
    )(colpart)

    n1 = T_out.shape[1]
    n2 = T_out.shape[2]
    return out1p[:n1], out2p[0, :n2]


# speedup vs baseline: 1.0683x; 1.0023x over previous
import functools

import jax
import jax.numpy as jnp
from jax import lax
from jax.experimental import pallas as pl
from jax.experimental.pallas import tpu as pltpu
from jax.experimental.pallas import tpu_sc as plsc

KP = 1_015_808
ROWS_P = 2048
COLS_P = 1024
NEG = -9999.0
BK = 32768
CH = 8192
L = 16


PADCELL = ROWS_P * COLS_P - 1


def _mlp_body(K, inp, idx, w1, b1, w2, b2, w3, b3, w4, b4, xo, co):
    a = inp[...]
    h = jnp.maximum(jnp.dot(w1[...], a, preferred_element_type=jnp.float32) + b1[...], 0.0)
    h = jnp.maximum(jnp.dot(w2[...], h, preferred_element_type=jnp.float32) + b2[...], 0.0)
    h = jnp.maximum(jnp.dot(w3[...], h, preferred_element_type=jnp.float32) + b3[...], 0.0)
    xo[...] = (jnp.dot(w4[...], h, preferred_element_type=jnp.float32) + b4[...]).reshape(BK)
    pos = pl.program_id(0) * BK + lax.broadcasted_iota(jnp.int32, (1, BK), 1)
    i01 = idx[...]
    cell = i01[0:1, :] * COLS_P + i01[1:2, :]
    co[...] = jnp.where(pos < K, cell, PADCELL).reshape(BK)


def _mlp_call(inp_p, idx, W1, b1, W2, b2, W3, b3, W4, b4):
    grid = KP // BK
    full = lambda i: (0, 0)
    lane = lambda i: (0, i)
    K = inp_p.shape[-1]
    return pl.pallas_call(
        functools.partial(_mlp_body, K),
        grid=(grid,),
        in_specs=[
            pl.BlockSpec((9, BK), lane),
            pl.BlockSpec((2, BK), lane),
            pl.BlockSpec((18, 9), full),
            pl.BlockSpec((18, 1), full),
            pl.BlockSpec((36, 18), full),
            pl.BlockSpec((36, 1), full),
            pl.BlockSpec((36, 36), full),
            pl.BlockSpec((36, 1), full),
            pl.BlockSpec((1, 36), full),
            pl.BlockSpec((1, 1), full),
        ],
        out_specs=[pl.BlockSpec((BK,), lambda i: (i,)),
                   pl.BlockSpec((BK,), lambda i: (i,))],
        out_shape=[
            jax.ShapeDtypeStruct((KP,), jnp.float32),
            jax.ShapeDtypeStruct((KP,), jnp.int32),
        ],
    )(inp_p, idx, W1, b1.reshape(18, 1), W2, b2.reshape(36, 1),
      W3, b3.reshape(36, 1), W4, b4.reshape(1, 1))


@functools.cache
def _make_sc_scatter(NC, NS):
    NW = NC * NS
    CELLS_PER = (ROWS_P * COLS_P) // NW
    NROWS = ROWS_P // NW
    NCH = KP // CH
    NB = 2
    mesh = plsc.VectorSubcoreMesh(core_axis_name="c", subcore_axis_name="s")

    @functools.partial(
        pl.kernel,
        out_type=(jax.ShapeDtypeStruct((ROWS_P,), jnp.float32),
                  jax.ShapeDtypeStruct((NW, COLS_P), jnp.float32)),
        mesh=mesh,
        scratch_types=[
            pltpu.VMEM((CELLS_PER,), jnp.float32),
            pltpu.VMEM((CH,), jnp.int32),
            pltpu.VMEM((CH,), jnp.int32),
            pltpu.VMEM((CH,), jnp.float32),
            pltpu.VMEM((CH,), jnp.float32),
            pltpu.VMEM((COLS_P,), jnp.float32),
            pltpu.VMEM((NROWS,), jnp.float32),
            pltpu.SemaphoreType.DMA,
            pltpu.SemaphoreType.DMA,
            pltpu.SemaphoreType.DMA,
            pltpu.SemaphoreType.DMA,
        ],
        compiler_params=pltpu.CompilerParams(needs_layout_passes=False),
    )
    def sc_a(cell_hbm, x_hbm, out1_hbm, colpart_hbm,
             vtab, cbuf0, cbuf1, xbuf0, xbuf1, colp, rowm,
             csem0, csem1, xsem0, xsem1):
        w = lax.axis_index("c") * NS + lax.axis_index("s")
        base = w * CELLS_PER
        iota = lax.iota(jnp.int32, L)
        negs = jnp.full((L,), NEG, jnp.float32)

        def init_body(i, _):
            vtab[pl.ds(i * L, L)] = negs
            return 0
        lax.fori_loop(0, CELLS_PER // L, init_body, 0)

        cbufs = (cbuf0, cbuf1)
        xbufs = (xbuf0, xbuf1)
        csems = (csem0, csem1)
        xsems = (xsem0, xsem1)
        for b in range(NB):
            pltpu.async_copy(cell_hbm.at[pl.ds(b * CH, CH)], cbufs[b], csems[b])
            pltpu.async_copy(x_hbm.at[pl.ds(b * CH, CH)], xbufs[b], xsems[b])

        UN = 8

        def process(cbuf, xbuf):
            def body(i, _):
                for u in range(UN):
                    j = i * UN + u
                    cv = cbuf[pl.ds(j * L, L)]
                    xv = xbuf[pl.ds(j * L, L)]
                    lv = cv - base
                    valid = lv.astype(jnp.uint32) < CELLS_PER
                    plsc.store_scatter(vtab, [lv], xv, mask=valid)
                return 0
            lax.fori_loop(0, CH // (L * UN), body, 0)

        def outer(g, _):
            for b in range(NB):
                ch = g * NB + b
                pltpu.make_async_copy(
                    cell_hbm.at[pl.ds(ch * CH, CH)], cbufs[b], csems[b]).wait()
                pltpu.make_async_copy(
                    x_hbm.at[pl.ds(ch * CH, CH)], xbufs[b], xsems[b]).wait()
                process(cbufs[b], xbufs[b])

                @pl.when(ch + NB < NCH)
                def _next():
                    pltpu.async_copy(
                        cell_hbm.at[pl.ds((ch + NB) * CH, CH)], cbufs[b], csems[b])
                    pltpu.async_copy(
                        x_hbm.at[pl.ds((ch + NB) * CH, CH)], xbufs[b], xsems[b])
            return 0
        lax.fori_loop(0, NCH // NB, outer, 0)

        def cinit(i, _):
            colp[pl.ds(i * L, L)] = negs
            return 0
        lax.fori_loop(0, COLS_P // L, cinit, 0)

        RUN = 4

        def row_body(r, _):
            def mbody(i, acc):
                accs = []
                for u in range(RUN):
                    j = i * RUN + u
                    vv = vtab[pl.ds(r * COLS_P + j * L, L)]
                    colp[pl.ds(j * L, L)] = jnp.maximum(colp[pl.ds(j * L, L)], vv)
                    accs.append(vv)
                m = jnp.maximum(jnp.maximum(accs[0], accs[1]),
                                jnp.maximum(accs[2], accs[3]))
                return jnp.maximum(acc, m)
            rm = lax.fori_loop(0, COLS_P // (L * RUN), mbody, negs)
            plsc.store_scatter(rowm, [jnp.full((L,), r, jnp.int32)],
                               jnp.full((L,), jnp.max(rm), jnp.float32),
                               mask=iota == 0)
            return 0
        lax.fori_loop(0, NROWS, row_body, 0)

        pltpu.sync_copy(rowm, out1_hbm.at[pl.ds(w * NROWS, NROWS)])
        pltpu.sync_copy(colp, colpart_hbm.at[w])

    return sc_a


def _colmax_body(cp, o):
    o[...] = jnp.max(cp[...], axis=0, keepdims=True)


def kernel(input, T_out, T_indices, W1, b1, W2, b2, W3, b3, W4, b4):
    K = input.shape[-1]
    inp_p = input.reshape(9, K)
    idx = T_indices.astype(jnp.int32)

    x, cells = _mlp_call(inp_p, idx, W1, b1, W2, b2, W3, b3, W4, b4)

    info = plsc.get_sparse_core_info()
    sc_a = _make_sc_scatter(info.num_cores, info.num_subcores)
    out1p, colpart = sc_a(cells, x)

    out2p = pl.pallas_call(
        _colmax_body,
        out_shape=jax.ShapeDtypeStruct((1, COLS_P), jnp.float32),
    )(colpart)

    n1 = T_out.shape[1]
    n2 = T_out.shape[2]
    return out1p[:n1], out2p[0, :n2]
